# baseline (speedup 1.0000x reference)
import jax
import jax.numpy as jnp
from jax.experimental import pallas as pl
from jax.experimental.pallas import tpu as pltpu

_ROWS = 2048


def _gelu_exact(x):
    return 0.5 * x * (1.0 + jax.lax.erf(x * 0.7071067811865476))


def _make_mlp_kernel(n_tiles, rows):
    def _mlp_kernel(x_hbm, w1_ref, b1_ref, w2_ref, b2_ref, w3_ref, b3_ref,
                    o_hbm, xbuf, obuf, in_sem, out_sem):
        def in_copy(i, slot):
            return pltpu.make_async_copy(
                x_hbm.at[pl.ds(i * rows, rows), :], xbuf.at[slot],
                in_sem.at[slot])

        def out_copy(i, slot):
            return pltpu.make_async_copy(
                obuf.at[slot], o_hbm.at[pl.ds(i * rows, rows), :],
                out_sem.at[slot])

        in_copy(0, 0).start()
        for i in range(n_tiles):
            slot = i % 2
            if i + 1 < n_tiles:
                in_copy(i + 1, 1 - slot).start()
            in_copy(i, slot).wait()
            x = xbuf[slot]
            h = jnp.dot(x, w1_ref[...], preferred_element_type=jnp.float32) + b1_ref[...]
            h = _gelu_exact(h)
            h = jnp.dot(h, w2_ref[...], preferred_element_type=jnp.float32) + b2_ref[...]
            h = _gelu_exact(h)
            res = jnp.dot(h, w3_ref[...], preferred_element_type=jnp.float32) + b3_ref[...]
            if i >= 2:
                out_copy(i - 2, slot).wait()
            obuf[slot] = res
            out_copy(i, slot).start()
        if n_tiles >= 2:
            out_copy(n_tiles - 2, (n_tiles - 2) % 2).wait()
        out_copy(n_tiles - 1, (n_tiles - 1) % 2).wait()

    return _mlp_kernel


def kernel(embedding, W1, b1, W2, b2, W3, b3):
    n, d = embedding.shape
    h = W1.shape[1]
    rows = min(_ROWS, n)
    n_tiles = n // rows
    vmem = pl.BlockSpec(memory_space=pltpu.VMEM)
    return pl.pallas_call(
        _make_mlp_kernel(n_tiles, rows),
        in_specs=[
            pl.BlockSpec(memory_space=pl.ANY),
            vmem, vmem, vmem, vmem, vmem, vmem,
        ],
        out_specs=pl.BlockSpec(memory_space=pl.ANY),
        out_shape=jax.ShapeDtypeStruct((n, d), jnp.float32),
        scratch_shapes=[
            pltpu.VMEM((2, rows, d), jnp.float32),
            pltpu.VMEM((2, rows, d), jnp.float32),
            pltpu.SemaphoreType.DMA((2,)),
            pltpu.SemaphoreType.DMA((2,)),
        ],
    )(embedding, W1, b1.reshape(1, h), W2, b2.reshape(1, h), W3, b3.reshape(1, d))


# speedup vs baseline: 1.1279x; 1.1279x over previous
import jax
import jax.numpy as jnp
from jax.experimental import pallas as pl
from jax.experimental.pallas import tpu as pltpu

_ROWS = 4096
_SUB = 2048


def _gelu_exact(x):
    return 0.5 * x * (1.0 + jax.lax.erf(x * 0.7071067811865476))


def _mlp_kernel(x_ref, w1_ref, b1_ref, w2_ref, b2_ref, w3_ref, b3_ref, o_ref):
    rows = x_ref.shape[0]
    for j in range(pl.cdiv(rows, _SUB)):
        sl = pl.ds(j * _SUB, min(_SUB, rows))
        x = x_ref[sl, :]
        h = jnp.dot(x, w1_ref[...], preferred_element_type=jnp.float32) + b1_ref[...]
        h = _gelu_exact(h)
        h = jnp.dot(h, w2_ref[...], preferred_element_type=jnp.float32) + b2_ref[...]
        h = _gelu_exact(h)
        o_ref[sl, :] = jnp.dot(h, w3_ref[...], preferred_element_type=jnp.float32) + b3_ref[...]


def kernel(embedding, W1, b1, W2, b2, W3, b3):
    n, d = embedding.shape
    h = W1.shape[1]
    rows = min(_ROWS, n)
    grid = (pl.cdiv(n, rows),)
    return pl.pallas_call(
        _mlp_kernel,
        grid=grid,
        in_specs=[
            pl.BlockSpec((rows, d), lambda i: (i, 0)),
            pl.BlockSpec((d, h), lambda i: (0, 0)),
            pl.BlockSpec((1, h), lambda i: (0, 0)),
            pl.BlockSpec((h, h), lambda i: (0, 0)),
            pl.BlockSpec((1, h), lambda i: (0, 0)),
            pl.BlockSpec((h, d), lambda i: (0, 0)),
            pl.BlockSpec((1, d), lambda i: (0, 0)),
        ],
        out_specs=pl.BlockSpec((rows, d), lambda i: (i, 0)),
        out_shape=jax.ShapeDtypeStruct((n, d), jnp.float32),
        compiler_params=pltpu.CompilerParams(
            dimension_semantics=("arbitrary",),
        ),
    )(embedding, W1, b1.reshape(1, h), W2, b2.reshape(1, h), W3, b3.reshape(1, d))


# speedup vs baseline: 1.1766x; 1.0432x over previous
import jax
import jax.numpy as jnp
from jax.experimental import pallas as pl
from jax.experimental.pallas import tpu as pltpu

_ROWS = 2048


def _gelu_exact(x):
    return 0.5 * x * (1.0 + jax.lax.erf(x * 0.7071067811865476))


def _mlp_kernel(xa_ref, xb_ref, w1_ref, b1_ref, w2_ref, b2_ref, w3_ref, b3_ref,
                o_ref):
    half = xa_ref.shape[0]
    for k, x_ref in enumerate((xa_ref, xb_ref)):
        x = x_ref[...]
        h = jnp.dot(x, w1_ref[...], preferred_element_type=jnp.float32) + b1_ref[...]
        h = _gelu_exact(h)
        h = jnp.dot(h, w2_ref[...], preferred_element_type=jnp.float32) + b2_ref[...]
        h = _gelu_exact(h)
        o_ref[pl.ds(k * half, half), :] = (
            jnp.dot(h, w3_ref[...], preferred_element_type=jnp.float32) + b3_ref[...])


def kernel(embedding, W1, b1, W2, b2, W3, b3):
    n, d = embedding.shape
    h = W1.shape[1]
    rows = min(_ROWS, n)
    half = rows // 2
    grid = (n // rows,)
    return pl.pallas_call(
        _mlp_kernel,
        grid=grid,
        in_specs=[
            pl.BlockSpec((half, d), lambda i: (2 * i, 0)),
            pl.BlockSpec((half, d), lambda i: (2 * i + 1, 0)),
            pl.BlockSpec((d, h), lambda i: (0, 0)),
            pl.BlockSpec((1, h), lambda i: (0, 0)),
            pl.BlockSpec((h, h), lambda i: (0, 0)),
            pl.BlockSpec((1, h), lambda i: (0, 0)),
            pl.BlockSpec((h, d), lambda i: (0, 0)),
            pl.BlockSpec((1, d), lambda i: (0, 0)),
        ],
        out_specs=pl.BlockSpec((rows, d), lambda i: (i, 0)),
        out_shape=jax.ShapeDtypeStruct((n, d), jnp.float32),
        compiler_params=pltpu.CompilerParams(
            dimension_semantics=("arbitrary",),
        ),
    )(embedding, embedding, W1, b1.reshape(1, h), W2, b2.reshape(1, h), W3,
      b3.reshape(1, d))


# speedup vs baseline: 1.1905x; 1.0118x over previous
import jax
import jax.numpy as jnp
from jax.experimental import pallas as pl
from jax.experimental.pallas import tpu as pltpu

_ROWS = 2048


def _gelu_exact(x):
    return 0.5 * x * (1.0 + jax.lax.erf(x * 0.7071067811865476))


def _mlp_kernel(x_ref, w1_ref, b1_ref, w2_ref, b2_ref, w3_ref, b3_ref, o_ref):
    x = x_ref[...]
    h = jnp.dot(x, w1_ref[...], preferred_element_type=jnp.float32) + b1_ref[...]
    h = _gelu_exact(h)
    h = jnp.dot(h, w2_ref[...], preferred_element_type=jnp.float32) + b2_ref[...]
    h = _gelu_exact(h)
    o_ref[...] = jnp.dot(h, w3_ref[...], preferred_element_type=jnp.float32) + b3_ref[...]


def kernel(embedding, W1, b1, W2, b2, W3, b3):
    n, d = embedding.shape
    h = W1.shape[1]
    rows = min(_ROWS, n)
    grid = (n // rows,)
    return pl.pallas_call(
        _mlp_kernel,
        grid=grid,
        in_specs=[
            pl.BlockSpec((rows, d), lambda i: (i, 0)),
            pl.BlockSpec((d, h), lambda i: (0, 0)),
            pl.BlockSpec((1, h), lambda i: (0, 0)),
            pl.BlockSpec((h, h), lambda i: (0, 0)),
            pl.BlockSpec((1, h), lambda i: (0, 0)),
            pl.BlockSpec((h, d), lambda i: (0, 0)),
            pl.BlockSpec((1, d), lambda i: (0, 0)),
        ],
        out_specs=pl.BlockSpec((rows, d), lambda i: (i, 0)),
        out_shape=jax.ShapeDtypeStruct((n, d), jnp.float32),
        compiler_params=pltpu.CompilerParams(
            dimension_semantics=("parallel",),
        ),
    )(embedding, W1, b1.reshape(1, h), W2, b2.reshape(1, h), W3, b3.reshape(1, d))
